# baseline (speedup 1.0000x reference)
import functools

import jax
import jax.numpy as jnp
from jax import lax
from jax.experimental import pallas as pl
from jax.experimental.pallas import tpu as pltpu
from jax.experimental.pallas import tpu_sc as plsc

_NC = 2
_NS = 16
_LANES = 16
_CH = 80


def _sc_segsum(feat_p, emb_p, src2, dst2):
    Np, D = feat_p.shape
    nrows, ch = src2.shape
    assert ch == _CH
    E = nrows * _CH
    epw = E // _NS
    nch = epw // _CH
    IB = 25
    nblk = nch // IB
    rows_pt = Np // _NS
    assert epw * _NS == E and nch * _CH == epw and rows_pt * _NS == Np
    assert nblk * IB == nch and rows_pt % _CH == 0

    mesh = plsc.VectorSubcoreMesh(core_axis_name="c", subcore_axis_name="s")

    @functools.partial(
        pl.kernel,
        out_type=[
            jax.ShapeDtypeStruct((Np, D), jnp.float32),
            jax.ShapeDtypeStruct((Np, D), jnp.float32),
            jax.ShapeDtypeStruct((Np,), jnp.float32),
        ],
        mesh=mesh,
        scratch_types=[
            pltpu.VMEM_SHARED((Np, D), jnp.float32),
            pltpu.VMEM_SHARED((Np,), jnp.float32),
            pltpu.VMEM((IB, _CH), jnp.int32),
            pltpu.VMEM((IB, _CH), jnp.int32),
            pltpu.VMEM((3, _CH, D), jnp.float32),
            pltpu.VMEM((_CH,), jnp.float32),
            pltpu.VMEM((Np,), jnp.float32),
            pltpu.SemaphoreType.DMA,
            pltpu.SemaphoreType.DMA,
            pltpu.SemaphoreType.DMA,
            pltpu.SemaphoreType.DMA,
            pltpu.SemaphoreType.DMA,
            pltpu.SemaphoreType.DMA,
            pltpu.SemaphoreType.DMA,
            pltpu.SemaphoreType.DMA,
            pltpu.SemaphoreType.DMA,
        ],
        compiler_params=pltpu.CompilerParams(use_tc_tiling_on_sc=False),
    )
    def k(feat_h, emb_h, src_h, dst_h, sfeat_h, semb_h, deg_h,
          acc_s, deg_s, srcv, dstv, rowsv, onesv, degv,
          gs0, gs1, gs2, ss0, ss1, ss2, ds0, ds1, ds2):
        cid = lax.axis_index("c")
        sid = lax.axis_index("s")
        zero16 = jnp.zeros((_LANES,), jnp.float32)
        one16 = jnp.ones((_LANES,), jnp.float32)
        gsem = (gs0, gs1, gs2)
        ssem = (ss0, ss1, ss2)
        dsem = (ds0, ds1, ds2)

        def zrow(i, _):
            for j in range(D // _LANES):
                rowsv[0, i, pl.ds(j * _LANES, _LANES)] = zero16
            return 0
        lax.fori_loop(0, _CH, zrow, 0)

        def zcp(t, _):
            off = pl.multiple_of(sid * rows_pt + t * _CH, 8)
            pltpu.sync_copy(rowsv.at[0], acc_s.at[pl.ds(off, _CH)])
            return 0
        lax.fori_loop(0, rows_pt // _CH, zcp, 0)

        for j in range(_CH // _LANES):
            onesv[pl.ds(j * _LANES, _LANES)] = one16

        @pl.when(sid == 0)
        def _():
            def zdeg(i, _):
                degv[pl.ds(i * _LANES, _LANES)] = zero16
                return 0
            lax.fori_loop(0, Np // _LANES, zdeg, 0)
            pltpu.sync_copy(degv, deg_s)

        plsc.subcore_barrier()

        def sweep(table_h, with_deg):
            def block(bi, _):
                irow = sid * nch + bi * IB
                pltpu.sync_copy(src_h.at[pl.ds(irow, IB)], srcv)
                pltpu.sync_copy(dst_h.at[pl.ds(irow, IB)], dstv)
                gath = {}
                scat = {}
                dadd = {}

                def fire_gather(j):
                    b = j % 3
                    gath[j] = pltpu.async_copy(
                        table_h.at[srcv.at[j]], rowsv.at[b], gsem[b])

                def fire_scatter(j):
                    b = j % 3
                    scat[j] = pltpu.async_copy(
                        rowsv.at[b], acc_s.at[dstv.at[j]], ssem[b], add=True)
                    if with_deg:
                        dadd[j] = pltpu.async_copy(
                            onesv, deg_s.at[dstv.at[j]], dsem[b], add=True)

                fire_gather(0)
                for j in range(IB):
                    if j >= 2:
                        scat[j - 2].wait()
                        if with_deg:
                            dadd[j - 2].wait()
                    if j + 1 < IB:
                        fire_gather(j + 1)
                    gath[j].wait()
                    fire_scatter(j)
                for j in (IB - 2, IB - 1):
                    scat[j].wait()
                    if with_deg:
                        dadd[j].wait()
                return 0
            lax.fori_loop(0, nblk, block, 0)

        @pl.when(cid == 0)
        def _():
            sweep(feat_h, True)

        @pl.when(cid == 1)
        def _():
            sweep(emb_h, False)

        plsc.subcore_barrier()

        def ocp(t, _):
            off = pl.multiple_of(sid * rows_pt + t * _CH, 8)
            pltpu.sync_copy(acc_s.at[pl.ds(off, _CH)], rowsv.at[0])

            @pl.when(cid == 0)
            def _():
                pltpu.sync_copy(rowsv.at[0], sfeat_h.at[pl.ds(off, _CH)])

            @pl.when(cid == 1)
            def _():
                pltpu.sync_copy(rowsv.at[0], semb_h.at[pl.ds(off, _CH)])

            return 0
        lax.fori_loop(0, rows_pt // _CH, ocp, 0)

        @pl.when(jnp.logical_and(cid == 0, sid == 0))
        def _():
            pltpu.sync_copy(deg_s, degv)
            pltpu.sync_copy(degv, deg_h)

    return k(feat_p, emb_p, src2, dst2)


def _tc_tables(feat_p, emb_p, sfeat, semb, degb, fcWt, fcb2, w1t, w2t, w3t, linb2):
    Np, D = feat_p.shape
    BLK = 512
    assert Np % BLK == 0

    def body(f_r, e_r, sf_r, se_r, dg_r, fw_r, fb_r, w1_r, w2_r, w3_r, lb_r,
             p12_r, q_r):
        deg = dg_r[...]
        mask = deg > 0.0
        degc = jnp.maximum(deg, 1.0)
        f = f_r[...]
        e = e_r[...]
        m1_in = jnp.dot(f + sf_r[...] / degc, fw_r[...],
                        preferred_element_type=jnp.float32) + 2.0 * fb_r[...]
        m1 = jnp.where(mask, jnp.maximum(m1_in, 0.0), 0.0)
        root1 = jnp.where(mask, se_r[...] / degc, e)
        p1 = jnp.dot(e, w1_r[...], preferred_element_type=jnp.float32)
        p2 = (jnp.dot(root1, w1_r[...], preferred_element_type=jnp.float32)
              + jnp.dot(m1, w2_r[...], preferred_element_type=jnp.float32))
        p12_r[...] = jnp.concatenate([p1, p2], axis=1)
        q_r[...] = jnp.dot(f, w3_r[...], preferred_element_type=jnp.float32) + lb_r[...]

    row_spec = pl.BlockSpec((BLK, D), lambda i: (i, 0))
    w_spec = pl.BlockSpec((D, D), lambda i: (0, 0))
    b_spec = pl.BlockSpec((1, D), lambda i: (0, 0))
    return pl.pallas_call(
        body,
        grid=(Np // BLK,),
        in_specs=[row_spec, row_spec, row_spec, row_spec, row_spec,
                  w_spec, b_spec, w_spec, w_spec, w_spec, b_spec],
        out_specs=[pl.BlockSpec((BLK, 2 * D), lambda i: (i, 0)), row_spec],
        out_shape=[
            jax.ShapeDtypeStruct((Np, 2 * D), jnp.float32),
            jax.ShapeDtypeStruct((Np, D), jnp.float32),
        ],
    )(feat_p, emb_p, sfeat, semb, degb, fcWt, fcb2, w1t, w2t, w3t, linb2)


def _sc_edge_head(p12, q, src2, dst2, u):
    Np, D2 = p12.shape
    D = q.shape[1]
    nrows, ch = src2.shape
    assert ch == _CH
    E = nrows * _CH
    NW = _NC * _NS
    nch = nrows // NW
    IBC = 5
    nblk = nch // IBC
    assert nch * NW == nrows and nblk * IBC == nch

    mesh = plsc.VectorSubcoreMesh(core_axis_name="c", subcore_axis_name="s")

    @functools.partial(
        pl.kernel,
        out_type=jax.ShapeDtypeStruct((2, nrows, _CH), jnp.float32),
        mesh=mesh,
        scratch_types=[
            pltpu.VMEM((IBC, _CH), jnp.int32),
            pltpu.VMEM((IBC, _CH), jnp.int32),
            pltpu.VMEM((3, _CH, D2), jnp.float32),
            pltpu.VMEM((3, _CH, D), jnp.float32),
            pltpu.VMEM((D,), jnp.float32),
            pltpu.VMEM((IBC, _CH), jnp.float32),
            pltpu.VMEM((IBC, _CH), jnp.float32),
            pltpu.VMEM((_LANES * _LANES,), jnp.float32),
            pltpu.VMEM((_LANES * _LANES,), jnp.float32),
            pltpu.SemaphoreType.DMA,
            pltpu.SemaphoreType.DMA,
            pltpu.SemaphoreType.DMA,
            pltpu.SemaphoreType.DMA,
            pltpu.SemaphoreType.DMA,
            pltpu.SemaphoreType.DMA,
        ],
        compiler_params=pltpu.CompilerParams(use_tc_tiling_on_sc=False,
                                            needs_layout_passes=False),
    )
    def k(p_h, q_h, src_h, dst_h, u_h, out_h,
          srcv, dstv, pv, qv, uv, o1v, o2v, amat1, amat2,
          ps0, ps1, ps2, qs0, qs1, qs2):
        cid = lax.axis_index("c")
        sid = lax.axis_index("s")
        wid = sid * _NC + cid
        psem = (ps0, ps1, ps2)
        qsem = (qs0, qs1, qs2)
        pltpu.sync_copy(u_h, uv)
        uregs = [uv[pl.ds(j * _LANES, _LANES)] for j in range(D // _LANES)]
        lane = lax.iota(jnp.int32, _LANES)
        sidx = lane * _LANES

        def block(bi, _):
            irow = wid * nch + bi * IBC
            pltpu.sync_copy(src_h.at[pl.ds(irow, IBC)], srcv)
            pltpu.sync_copy(dst_h.at[pl.ds(irow, IBC)], dstv)
            gp = {}
            gq = {}

            def fire(j):
                b = j % 3
                gp[j] = pltpu.async_copy(p_h.at[srcv.at[j]], pv.at[b], psem[b])
                gq[j] = pltpu.async_copy(q_h.at[dstv.at[j]], qv.at[b], qsem[b])

            fire(0)
            fire(1)
            for j in range(IBC):
                b = j % 3
                if j + 2 < IBC:
                    fire(j + 2)
                gp[j].wait()
                gq[j].wait()

                def group(g, _, b=b, j=j):
                    row0 = pl.multiple_of(g * _LANES, 8)
                    for e16 in range(_LANES):
                        r = row0 + e16
                        qr = qv.at[b, r]
                        pr = pv.at[b, r]
                        a1 = jnp.zeros((_LANES,), jnp.float32)
                        a2 = jnp.zeros((_LANES,), jnp.float32)
                        for jj in range(D // _LANES):
                            qj = qr[pl.ds(jj * _LANES, _LANES)]
                            p1 = pr[pl.ds(jj * _LANES, _LANES)]
                            p2 = pr[pl.ds(D + jj * _LANES, _LANES)]
                            a1 = a1 + jnp.maximum(p1 + qj, 0.0) * uregs[jj]
                            a2 = a2 + jnp.maximum(p2 + qj, 0.0) * uregs[jj]
                        plsc.store_scatter(amat1, [sidx + e16], a1)
                        plsc.store_scatter(amat2, [sidx + e16], a2)
                    s1 = amat1[pl.ds(0, _LANES)]
                    s2 = amat2[pl.ds(0, _LANES)]
                    for l in range(1, _LANES):
                        s1 = s1 + amat1[pl.ds(l * _LANES, _LANES)]
                        s2 = s2 + amat2[pl.ds(l * _LANES, _LANES)]
                    o1v[j, pl.ds(row0, _LANES)] = s1
                    o2v[j, pl.ds(row0, _LANES)] = s2
                    return 0
                lax.fori_loop(0, _CH // _LANES, group, 0)

            pltpu.sync_copy(o1v, out_h.at[0, pl.ds(irow, IBC)])
            pltpu.sync_copy(o2v, out_h.at[1, pl.ds(irow, IBC)])
            return 0
        lax.fori_loop(0, nblk, block, 0)

    return k(p12, q, src2, dst2, u)


def kernel(features, emb, fc_W, fc_b, lin_W, lin_b, us_W, us_b, edge_index):
    N, D = features.shape
    H = emb.shape[1]
    E = edge_index.shape[1]
    Np = ((N + 511) // 512) * 512

    src2 = edge_index[0].reshape(E // _CH, _CH)
    dst2 = edge_index[1].reshape(E // _CH, _CH)
    feat_p = jnp.pad(features, ((0, Np - N), (0, 0)))
    emb_p = jnp.pad(emb, ((0, Np - N), (0, 0)))

    sfeat, semb, deg = _sc_segsum(feat_p, emb_p, src2, dst2)
    degb = jnp.broadcast_to(deg[:, None], (Np, D))

    w1t = lin_W[:, :H].T
    w2t = lin_W[:, H:2 * H].T
    w3t = lin_W[:, 2 * H:].T
    p12, q = _tc_tables(feat_p, emb_p, sfeat, semb, degb,
                        fc_W.T, fc_b[None, :], w1t, w2t, w3t, lin_b[None, :])

    out3 = _sc_edge_head(p12, q, src2, dst2, us_W[0])
    return out3.reshape(2, E, 1) + us_b


# speedup vs baseline: 1.3306x; 1.3306x over previous
import functools

import jax
import jax.numpy as jnp
from jax import lax
from jax.experimental import pallas as pl
from jax.experimental.pallas import tpu as pltpu
from jax.experimental.pallas import tpu_sc as plsc

_NC = 2
_NS = 16
_LANES = 16
_CH = 80


def _sc_segsum(feat_p, emb_p, src2, dst2):
    Np, D = feat_p.shape
    nrows, ch = src2.shape
    assert ch == _CH
    E = nrows * _CH
    epw = E // _NS
    nch = epw // _CH
    IB = 25
    nblk = nch // IB
    rows_pt = Np // _NS
    assert epw * _NS == E and nch * _CH == epw and rows_pt * _NS == Np
    assert nblk * IB == nch and rows_pt % _CH == 0

    mesh = plsc.VectorSubcoreMesh(core_axis_name="c", subcore_axis_name="s")

    @functools.partial(
        pl.kernel,
        out_type=[
            jax.ShapeDtypeStruct((Np, D), jnp.float32),
            jax.ShapeDtypeStruct((Np, D), jnp.float32),
            jax.ShapeDtypeStruct((Np,), jnp.float32),
        ],
        mesh=mesh,
        scratch_types=[
            pltpu.VMEM_SHARED((Np, D), jnp.float32),
            pltpu.VMEM_SHARED((Np,), jnp.float32),
            pltpu.VMEM((IB, _CH), jnp.int32),
            pltpu.VMEM((IB, _CH), jnp.int32),
            pltpu.VMEM((3, _CH, D), jnp.float32),
            pltpu.VMEM((_CH,), jnp.float32),
            pltpu.VMEM((Np,), jnp.float32),
            pltpu.SemaphoreType.DMA,
            pltpu.SemaphoreType.DMA,
            pltpu.SemaphoreType.DMA,
            pltpu.SemaphoreType.DMA,
            pltpu.SemaphoreType.DMA,
            pltpu.SemaphoreType.DMA,
            pltpu.SemaphoreType.DMA,
            pltpu.SemaphoreType.DMA,
            pltpu.SemaphoreType.DMA,
        ],
        compiler_params=pltpu.CompilerParams(use_tc_tiling_on_sc=False),
    )
    def k(feat_h, emb_h, src_h, dst_h, sfeat_h, semb_h, deg_h,
          acc_s, deg_s, srcv, dstv, rowsv, onesv, degv,
          gs0, gs1, gs2, ss0, ss1, ss2, ds0, ds1, ds2):
        cid = lax.axis_index("c")
        sid = lax.axis_index("s")
        zero16 = jnp.zeros((_LANES,), jnp.float32)
        one16 = jnp.ones((_LANES,), jnp.float32)
        gsem = (gs0, gs1, gs2)
        ssem = (ss0, ss1, ss2)
        dsem = (ds0, ds1, ds2)

        def zrow(i, _):
            for j in range(D // _LANES):
                rowsv[0, i, pl.ds(j * _LANES, _LANES)] = zero16
            return 0
        lax.fori_loop(0, _CH, zrow, 0)

        def zcp(t, _):
            off = pl.multiple_of(sid * rows_pt + t * _CH, 8)
            pltpu.sync_copy(rowsv.at[0], acc_s.at[pl.ds(off, _CH)])
            return 0
        lax.fori_loop(0, rows_pt // _CH, zcp, 0)

        for j in range(_CH // _LANES):
            onesv[pl.ds(j * _LANES, _LANES)] = one16

        @pl.when(sid == 0)
        def _():
            def zdeg(i, _):
                degv[pl.ds(i * _LANES, _LANES)] = zero16
                return 0
            lax.fori_loop(0, Np // _LANES, zdeg, 0)
            pltpu.sync_copy(degv, deg_s)

        plsc.subcore_barrier()

        def sweep(table_h, with_deg):
            def block(bi, _):
                irow = sid * nch + bi * IB
                pltpu.sync_copy(src_h.at[pl.ds(irow, IB)], srcv)
                pltpu.sync_copy(dst_h.at[pl.ds(irow, IB)], dstv)
                gath = {}
                scat = {}
                dadd = {}

                def fire_gather(j):
                    b = j % 3
                    gath[j] = pltpu.async_copy(
                        table_h.at[srcv.at[j]], rowsv.at[b], gsem[b])

                def fire_scatter(j):
                    b = j % 3
                    scat[j] = pltpu.async_copy(
                        rowsv.at[b], acc_s.at[dstv.at[j]], ssem[b], add=True)
                    if with_deg:
                        dadd[j] = pltpu.async_copy(
                            onesv, deg_s.at[dstv.at[j]], dsem[b], add=True)

                fire_gather(0)
                for j in range(IB):
                    if j >= 2:
                        scat[j - 2].wait()
                        if with_deg:
                            dadd[j - 2].wait()
                    if j + 1 < IB:
                        fire_gather(j + 1)
                    gath[j].wait()
                    fire_scatter(j)
                for j in (IB - 2, IB - 1):
                    scat[j].wait()
                    if with_deg:
                        dadd[j].wait()
                return 0
            lax.fori_loop(0, nblk, block, 0)

        @pl.when(cid == 0)
        def _():
            sweep(feat_h, True)

        @pl.when(cid == 1)
        def _():
            sweep(emb_h, False)

        plsc.subcore_barrier()

        def ocp(t, _):
            off = pl.multiple_of(sid * rows_pt + t * _CH, 8)
            pltpu.sync_copy(acc_s.at[pl.ds(off, _CH)], rowsv.at[0])

            @pl.when(cid == 0)
            def _():
                pltpu.sync_copy(rowsv.at[0], sfeat_h.at[pl.ds(off, _CH)])

            @pl.when(cid == 1)
            def _():
                pltpu.sync_copy(rowsv.at[0], semb_h.at[pl.ds(off, _CH)])

            return 0
        lax.fori_loop(0, rows_pt // _CH, ocp, 0)

        @pl.when(jnp.logical_and(cid == 0, sid == 0))
        def _():
            pltpu.sync_copy(deg_s, degv)
            pltpu.sync_copy(degv, deg_h)

    return k(feat_p, emb_p, src2, dst2)


def _tc_tables(feat_p, emb_p, sfeat, semb, degb, fcWt, fcb2, w1t, w2t, w3t, linb2):
    Np, D = feat_p.shape
    BLK = 512
    assert Np % BLK == 0

    def body(f_r, e_r, sf_r, se_r, dg_r, fw_r, fb_r, w1_r, w2_r, w3_r, lb_r,
             p12_r, q_r):
        deg = dg_r[...]
        mask = deg > 0.0
        degc = jnp.maximum(deg, 1.0)
        f = f_r[...]
        e = e_r[...]
        m1_in = jnp.dot(f + sf_r[...] / degc, fw_r[...],
                        preferred_element_type=jnp.float32) + 2.0 * fb_r[...]
        m1 = jnp.where(mask, jnp.maximum(m1_in, 0.0), 0.0)
        root1 = jnp.where(mask, se_r[...] / degc, e)
        p1 = jnp.dot(e, w1_r[...], preferred_element_type=jnp.float32)
        p2 = (jnp.dot(root1, w1_r[...], preferred_element_type=jnp.float32)
              + jnp.dot(m1, w2_r[...], preferred_element_type=jnp.float32))
        p12_r[...] = jnp.concatenate([p1, p2], axis=1).astype(jnp.bfloat16)
        q_r[...] = (jnp.dot(f, w3_r[...], preferred_element_type=jnp.float32)
                    + lb_r[...]).astype(jnp.bfloat16)

    row_spec = pl.BlockSpec((BLK, D), lambda i: (i, 0))
    w_spec = pl.BlockSpec((D, D), lambda i: (0, 0))
    b_spec = pl.BlockSpec((1, D), lambda i: (0, 0))
    return pl.pallas_call(
        body,
        grid=(Np // BLK,),
        in_specs=[row_spec, row_spec, row_spec, row_spec, row_spec,
                  w_spec, b_spec, w_spec, w_spec, w_spec, b_spec],
        out_specs=[pl.BlockSpec((BLK, 2 * D), lambda i: (i, 0)), row_spec],
        out_shape=[
            jax.ShapeDtypeStruct((Np, 2 * D), jnp.bfloat16),
            jax.ShapeDtypeStruct((Np, D), jnp.bfloat16),
        ],
    )(feat_p, emb_p, sfeat, semb, degb, fcWt, fcb2, w1t, w2t, w3t, linb2)


def _sc_edge_head(p12, q, src2, dst2, u):
    Np, D2 = p12.shape
    D = q.shape[1]
    nrows, ch = src2.shape
    assert ch == _CH
    E = nrows * _CH
    NW = _NC * _NS
    nch = nrows // NW
    IBC = 5
    nblk = nch // IBC
    assert nch * NW == nrows and nblk * IBC == nch

    mesh = plsc.VectorSubcoreMesh(core_axis_name="c", subcore_axis_name="s")

    @functools.partial(
        pl.kernel,
        out_type=jax.ShapeDtypeStruct((2, nrows, _CH), jnp.float32),
        mesh=mesh,
        scratch_types=[
            pltpu.VMEM((IBC, _CH), jnp.int32),
            pltpu.VMEM((IBC, _CH), jnp.int32),
            pltpu.VMEM((3, _CH, D2), jnp.bfloat16),
            pltpu.VMEM((3, _CH, D), jnp.bfloat16),
            pltpu.VMEM((D,), jnp.float32),
            pltpu.VMEM((IBC, _CH), jnp.float32),
            pltpu.VMEM((IBC, _CH), jnp.float32),
            pltpu.SemaphoreType.DMA,
            pltpu.SemaphoreType.DMA,
            pltpu.SemaphoreType.DMA,
            pltpu.SemaphoreType.DMA,
            pltpu.SemaphoreType.DMA,
            pltpu.SemaphoreType.DMA,
        ],
        compiler_params=pltpu.CompilerParams(use_tc_tiling_on_sc=False,
                                            needs_layout_passes=False),
    )
    def k(p_h, q_h, src_h, dst_h, u_h, out_h,
          srcv, dstv, pv, qv, uv, o1v, o2v,
          ps0, ps1, ps2, qs0, qs1, qs2):
        cid = lax.axis_index("c")
        sid = lax.axis_index("s")
        wid = sid * _NC + cid
        psem = (ps0, ps1, ps2)
        qsem = (qs0, qs1, qs2)
        pltpu.sync_copy(u_h, uv)
        uregs = [uv[pl.ds(j * _LANES, _LANES)] for j in range(D // _LANES)]
        lane = lax.iota(jnp.int32, _LANES)

        def block(bi, _):
            irow = wid * nch + bi * IBC
            pltpu.sync_copy(src_h.at[pl.ds(irow, IBC)], srcv)
            pltpu.sync_copy(dst_h.at[pl.ds(irow, IBC)], dstv)
            gp = {}
            gq = {}

            def fire(j):
                b = j % 3
                gp[j] = pltpu.async_copy(p_h.at[srcv.at[j]], pv.at[b], psem[b])
                gq[j] = pltpu.async_copy(q_h.at[dstv.at[j]], qv.at[b], qsem[b])

            fire(0)
            fire(1)
            for j in range(IBC):
                b = j % 3
                if j + 2 < IBC:
                    fire(j + 2)
                gp[j].wait()
                gq[j].wait()

                def group(g, _, b=b, j=j):
                    row0 = pl.multiple_of(g * _LANES, 8)
                    res1 = jnp.zeros((_LANES,), jnp.float32)
                    res2 = jnp.zeros((_LANES,), jnp.float32)
                    W = 2 * _LANES
                    for e16 in range(_LANES):
                        r = row0 + e16
                        qr = qv.at[b, r]
                        pr = pv.at[b, r]
                        a1 = jnp.zeros((_LANES,), jnp.float32)
                        a2 = jnp.zeros((_LANES,), jnp.float32)
                        for t in range(D // W):
                            qa, qb = plsc.unpack(qr[pl.ds(t * W, W)],
                                                 format=plsc.PackFormat.INTERLEAVED)
                            pa, pb = plsc.unpack(pr[pl.ds(t * W, W)],
                                                 format=plsc.PackFormat.INTERLEAVED)
                            ra, rb = plsc.unpack(pr[pl.ds(D + t * W, W)],
                                                 format=plsc.PackFormat.INTERLEAVED)
                            ue = uregs[2 * t]
                            uo = uregs[2 * t + 1]
                            a1 = a1 + jnp.maximum(pa + qa, 0.0) * ue
                            a1 = a1 + jnp.maximum(pb + qb, 0.0) * uo
                            a2 = a2 + jnp.maximum(ra + qa, 0.0) * ue
                            a2 = a2 + jnp.maximum(rb + qb, 0.0) * uo
                        res1 = jnp.where(lane == e16, jnp.sum(a1), res1)
                        res2 = jnp.where(lane == e16, jnp.sum(a2), res2)
                    o1v[j, pl.ds(row0, _LANES)] = res1
                    o2v[j, pl.ds(row0, _LANES)] = res2
                    return 0
                lax.fori_loop(0, _CH // _LANES, group, 0)

            pltpu.sync_copy(o1v, out_h.at[0, pl.ds(irow, IBC)])
            pltpu.sync_copy(o2v, out_h.at[1, pl.ds(irow, IBC)])
            return 0
        lax.fori_loop(0, nblk, block, 0)

    return k(p12, q, src2, dst2, u)


def kernel(features, emb, fc_W, fc_b, lin_W, lin_b, us_W, us_b, edge_index):
    N, D = features.shape
    H = emb.shape[1]
    E = edge_index.shape[1]
    Np = ((N + 511) // 512) * 512

    src2 = edge_index[0].reshape(E // _CH, _CH)
    dst2 = edge_index[1].reshape(E // _CH, _CH)
    feat_p = jnp.pad(features, ((0, Np - N), (0, 0)))
    emb_p = jnp.pad(emb, ((0, Np - N), (0, 0)))

    sfeat, semb, deg = _sc_segsum(feat_p, emb_p, src2, dst2)
    degb = jnp.broadcast_to(deg[:, None], (Np, D))

    w1t = lin_W[:, :H].T
    w2t = lin_W[:, H:2 * H].T
    w3t = lin_W[:, 2 * H:].T
    p12, q = _tc_tables(feat_p, emb_p, sfeat, semb, degb,
                        fc_W.T, fc_b[None, :], w1t, w2t, w3t, lin_b[None, :])

    u_de = us_W[0].reshape(D // 32, 16, 2).transpose(0, 2, 1).reshape(D)
    out3 = _sc_edge_head(p12, q, src2, dst2, u_de)
    return out3.reshape(2, E, 1) + us_b


# speedup vs baseline: 1.3339x; 1.0025x over previous
import functools

import jax
import jax.numpy as jnp
from jax import lax
from jax.experimental import pallas as pl
from jax.experimental.pallas import tpu as pltpu
from jax.experimental.pallas import tpu_sc as plsc

_NC = 2
_NS = 16
_LANES = 16
_CH = 80


def _sc_segsum(feat_p, emb_p, src2, dst2):
    Np, D = feat_p.shape
    nrows, ch = src2.shape
    assert ch == _CH
    E = nrows * _CH
    epw = E // _NS
    nch = epw // _CH
    IB = 25
    nblk = nch // IB
    rows_pt = Np // _NS
    assert epw * _NS == E and nch * _CH == epw and rows_pt * _NS == Np
    assert nblk * IB == nch and rows_pt % _CH == 0

    mesh = plsc.VectorSubcoreMesh(core_axis_name="c", subcore_axis_name="s")

    @functools.partial(
        pl.kernel,
        out_type=[
            jax.ShapeDtypeStruct((Np, D), jnp.float32),
            jax.ShapeDtypeStruct((Np, D), jnp.float32),
            jax.ShapeDtypeStruct((Np,), jnp.float32),
        ],
        mesh=mesh,
        scratch_types=[
            pltpu.VMEM_SHARED((Np, D), jnp.float32),
            pltpu.VMEM_SHARED((Np,), jnp.float32),
            pltpu.VMEM((IB, _CH), jnp.int32),
            pltpu.VMEM((IB, _CH), jnp.int32),
            pltpu.VMEM((3, _CH, D), jnp.float32),
            pltpu.VMEM((_CH,), jnp.float32),
            pltpu.VMEM((Np,), jnp.float32),
            pltpu.SemaphoreType.DMA,
            pltpu.SemaphoreType.DMA,
            pltpu.SemaphoreType.DMA,
            pltpu.SemaphoreType.DMA,
            pltpu.SemaphoreType.DMA,
            pltpu.SemaphoreType.DMA,
            pltpu.SemaphoreType.DMA,
            pltpu.SemaphoreType.DMA,
            pltpu.SemaphoreType.DMA,
        ],
        compiler_params=pltpu.CompilerParams(use_tc_tiling_on_sc=False),
    )
    def k(feat_h, emb_h, src_h, dst_h, sfeat_h, semb_h, deg_h,
          acc_s, deg_s, srcv, dstv, rowsv, onesv, degv,
          gs0, gs1, gs2, ss0, ss1, ss2, ds0, ds1, ds2):
        cid = lax.axis_index("c")
        sid = lax.axis_index("s")
        zero16 = jnp.zeros((_LANES,), jnp.float32)
        one16 = jnp.ones((_LANES,), jnp.float32)
        gsem = (gs0, gs1, gs2)
        ssem = (ss0, ss1, ss2)
        dsem = (ds0, ds1, ds2)

        def zrow(i, _):
            for j in range(D // _LANES):
                rowsv[0, i, pl.ds(j * _LANES, _LANES)] = zero16
            return 0
        lax.fori_loop(0, _CH, zrow, 0)

        def zcp(t, _):
            off = pl.multiple_of(sid * rows_pt + t * _CH, 8)
            pltpu.sync_copy(rowsv.at[0], acc_s.at[pl.ds(off, _CH)])
            return 0
        lax.fori_loop(0, rows_pt // _CH, zcp, 0)

        for j in range(_CH // _LANES):
            onesv[pl.ds(j * _LANES, _LANES)] = one16

        @pl.when(sid == 0)
        def _():
            def zdeg(i, _):
                degv[pl.ds(i * _LANES, _LANES)] = zero16
                return 0
            lax.fori_loop(0, Np // _LANES, zdeg, 0)
            pltpu.sync_copy(degv, deg_s)

        plsc.subcore_barrier()

        def sweep(table_h, with_deg):
            def block(bi, _):
                irow = sid * nch + bi * IB
                pltpu.sync_copy(src_h.at[pl.ds(irow, IB)], srcv)
                pltpu.sync_copy(dst_h.at[pl.ds(irow, IB)], dstv)
                gath = {}
                scat = {}
                dadd = {}

                def fire_gather(j):
                    b = j % 3
                    gath[j] = pltpu.async_copy(
                        table_h.at[srcv.at[j]], rowsv.at[b], gsem[b])

                def fire_scatter(j):
                    b = j % 3
                    scat[j] = pltpu.async_copy(
                        rowsv.at[b], acc_s.at[dstv.at[j]], ssem[b], add=True)
                    if with_deg:
                        dadd[j] = pltpu.async_copy(
                            onesv, deg_s.at[dstv.at[j]], dsem[b], add=True)

                fire_gather(0)
                for j in range(IB):
                    if j >= 2:
                        scat[j - 2].wait()
                        if with_deg:
                            dadd[j - 2].wait()
                    if j + 1 < IB:
                        fire_gather(j + 1)
                    gath[j].wait()
                    fire_scatter(j)
                for j in (IB - 2, IB - 1):
                    scat[j].wait()
                    if with_deg:
                        dadd[j].wait()
                return 0
            lax.fori_loop(0, nblk, block, 0)

        @pl.when(cid == 0)
        def _():
            sweep(feat_h, True)

        @pl.when(cid == 1)
        def _():
            sweep(emb_h, False)

        plsc.subcore_barrier()

        def ocp(t, _):
            off = pl.multiple_of(sid * rows_pt + t * _CH, 8)
            pltpu.sync_copy(acc_s.at[pl.ds(off, _CH)], rowsv.at[0])

            @pl.when(cid == 0)
            def _():
                pltpu.sync_copy(rowsv.at[0], sfeat_h.at[pl.ds(off, _CH)])

            @pl.when(cid == 1)
            def _():
                pltpu.sync_copy(rowsv.at[0], semb_h.at[pl.ds(off, _CH)])

            return 0
        lax.fori_loop(0, rows_pt // _CH, ocp, 0)

        @pl.when(jnp.logical_and(cid == 0, sid == 0))
        def _():
            pltpu.sync_copy(deg_s, degv)
            pltpu.sync_copy(degv, deg_h)

    return k(feat_p, emb_p, src2, dst2)


def _tc_tables(feat_p, emb_p, sfeat, semb, degb, fcWt, fcb2, w1t, w2t, w3t, linb2):
    Np, D = feat_p.shape
    BLK = 512
    assert Np % BLK == 0

    def body(f_r, e_r, sf_r, se_r, dg_r, fw_r, fb_r, w1_r, w2_r, w3_r, lb_r,
             p12_r, q_r):
        deg = dg_r[...]
        mask = deg > 0.0
        degc = jnp.maximum(deg, 1.0)
        f = f_r[...]
        e = e_r[...]
        m1_in = jnp.dot(f + sf_r[...] / degc, fw_r[...],
                        preferred_element_type=jnp.float32) + 2.0 * fb_r[...]
        m1 = jnp.where(mask, jnp.maximum(m1_in, 0.0), 0.0)
        root1 = jnp.where(mask, se_r[...] / degc, e)
        p1 = jnp.dot(e, w1_r[...], preferred_element_type=jnp.float32)
        p2 = (jnp.dot(root1, w1_r[...], preferred_element_type=jnp.float32)
              + jnp.dot(m1, w2_r[...], preferred_element_type=jnp.float32))
        p12_r[...] = jnp.concatenate([p1, p2], axis=1).astype(jnp.bfloat16)
        q_r[...] = (jnp.dot(f, w3_r[...], preferred_element_type=jnp.float32)
                    + lb_r[...]).astype(jnp.bfloat16)

    row_spec = pl.BlockSpec((BLK, D), lambda i: (i, 0))
    w_spec = pl.BlockSpec((D, D), lambda i: (0, 0))
    b_spec = pl.BlockSpec((1, D), lambda i: (0, 0))
    return pl.pallas_call(
        body,
        grid=(Np // BLK,),
        in_specs=[row_spec, row_spec, row_spec, row_spec, row_spec,
                  w_spec, b_spec, w_spec, w_spec, w_spec, b_spec],
        out_specs=[pl.BlockSpec((BLK, 2 * D), lambda i: (i, 0)), row_spec],
        out_shape=[
            jax.ShapeDtypeStruct((Np, 2 * D), jnp.bfloat16),
            jax.ShapeDtypeStruct((Np, D), jnp.bfloat16),
        ],
    )(feat_p, emb_p, sfeat, semb, degb, fcWt, fcb2, w1t, w2t, w3t, linb2)


def _sc_edge_head(p12, q, src2, dst2, u):
    Np, D2 = p12.shape
    D = q.shape[1]
    nrows, ch = src2.shape
    assert ch == _CH
    E = nrows * _CH
    NW = _NC * _NS
    nch = nrows // NW
    IBC = 5
    nblk = nch // IBC
    assert nch * NW == nrows and nblk * IBC == nch

    mesh = plsc.VectorSubcoreMesh(core_axis_name="c", subcore_axis_name="s")

    @functools.partial(
        pl.kernel,
        out_type=jax.ShapeDtypeStruct((2, nrows, _CH), jnp.float32),
        mesh=mesh,
        scratch_types=[
            pltpu.VMEM((IBC, _CH), jnp.int32),
            pltpu.VMEM((IBC, _CH), jnp.int32),
            pltpu.VMEM((3, _CH, D2), jnp.bfloat16),
            pltpu.VMEM((3, _CH, D), jnp.bfloat16),
            pltpu.VMEM((D,), jnp.float32),
            pltpu.VMEM((IBC, _CH), jnp.float32),
            pltpu.VMEM((IBC, _CH), jnp.float32),
            pltpu.SemaphoreType.DMA,
            pltpu.SemaphoreType.DMA,
            pltpu.SemaphoreType.DMA,
            pltpu.SemaphoreType.DMA,
            pltpu.SemaphoreType.DMA,
            pltpu.SemaphoreType.DMA,
        ],
        compiler_params=pltpu.CompilerParams(use_tc_tiling_on_sc=False,
                                            needs_layout_passes=False),
    )
    def k(p_h, q_h, src_h, dst_h, u_h, out_h,
          srcv, dstv, pv, qv, uv, o1v, o2v,
          ps0, ps1, ps2, qs0, qs1, qs2):
        cid = lax.axis_index("c")
        sid = lax.axis_index("s")
        wid = sid * _NC + cid
        psem = (ps0, ps1, ps2)
        qsem = (qs0, qs1, qs2)
        pltpu.sync_copy(u_h, uv)
        uregs = [uv[pl.ds(j * _LANES, _LANES)] for j in range(D // _LANES)]
        lane = lax.iota(jnp.int32, _LANES)

        def block(bi, _):
            irow = wid * nch + bi * IBC
            pltpu.sync_copy(src_h.at[pl.ds(irow, IBC)], srcv)
            pltpu.sync_copy(dst_h.at[pl.ds(irow, IBC)], dstv)
            gp = {}
            gq = {}

            def fire(j):
                b = j % 3
                gp[j] = pltpu.async_copy(p_h.at[srcv.at[j]], pv.at[b], psem[b])
                gq[j] = pltpu.async_copy(q_h.at[dstv.at[j]], qv.at[b], qsem[b])

            fire(0)
            fire(1)
            for j in range(IBC):
                b = j % 3
                if j + 2 < IBC:
                    fire(j + 2)
                gp[j].wait()
                gq[j].wait()

                def group(g, _, b=b, j=j):
                    row0 = pl.multiple_of(g * _LANES, 8)
                    res1 = jnp.zeros((_LANES,), jnp.float32)
                    res2 = jnp.zeros((_LANES,), jnp.float32)
                    W = 2 * _LANES
                    for e16 in range(_LANES):
                        r = row0 + e16
                        qr = qv.at[b, r]
                        pr = pv.at[b, r]
                        t1 = []
                        t2 = []
                        for t in range(D // W):
                            qa, qb = plsc.unpack(qr[pl.ds(t * W, W)],
                                                 format=plsc.PackFormat.INTERLEAVED)
                            pa, pb = plsc.unpack(pr[pl.ds(t * W, W)],
                                                 format=plsc.PackFormat.INTERLEAVED)
                            ra, rb = plsc.unpack(pr[pl.ds(D + t * W, W)],
                                                 format=plsc.PackFormat.INTERLEAVED)
                            ue = uregs[2 * t]
                            uo = uregs[2 * t + 1]
                            t1.append(jnp.maximum(pa + qa, 0.0) * ue)
                            t1.append(jnp.maximum(pb + qb, 0.0) * uo)
                            t2.append(jnp.maximum(ra + qa, 0.0) * ue)
                            t2.append(jnp.maximum(rb + qb, 0.0) * uo)
                        while len(t1) > 1:
                            t1 = [t1[i] + t1[i + 1] for i in range(0, len(t1), 2)]
                            t2 = [t2[i] + t2[i + 1] for i in range(0, len(t2), 2)]
                        res1 = jnp.where(lane == e16, jnp.sum(t1[0]), res1)
                        res2 = jnp.where(lane == e16, jnp.sum(t2[0]), res2)
                    o1v[j, pl.ds(row0, _LANES)] = res1
                    o2v[j, pl.ds(row0, _LANES)] = res2
                    return 0
                lax.fori_loop(0, _CH // _LANES, group, 0)

            pltpu.sync_copy(o1v, out_h.at[0, pl.ds(irow, IBC)])
            pltpu.sync_copy(o2v, out_h.at[1, pl.ds(irow, IBC)])
            return 0
        lax.fori_loop(0, nblk, block, 0)

    return k(p12, q, src2, dst2, u)


def kernel(features, emb, fc_W, fc_b, lin_W, lin_b, us_W, us_b, edge_index):
    N, D = features.shape
    H = emb.shape[1]
    E = edge_index.shape[1]
    Np = ((N + 511) // 512) * 512

    src2 = edge_index[0].reshape(E // _CH, _CH)
    dst2 = edge_index[1].reshape(E // _CH, _CH)
    feat_p = jnp.pad(features, ((0, Np - N), (0, 0)))
    emb_p = jnp.pad(emb, ((0, Np - N), (0, 0)))

    sfeat, semb, deg = _sc_segsum(feat_p, emb_p, src2, dst2)
    degb = jnp.broadcast_to(deg[:, None], (Np, D))

    w1t = lin_W[:, :H].T
    w2t = lin_W[:, H:2 * H].T
    w3t = lin_W[:, 2 * H:].T
    p12, q = _tc_tables(feat_p, emb_p, sfeat, semb, degb,
                        fc_W.T, fc_b[None, :], w1t, w2t, w3t, lin_b[None, :])

    u_de = us_W[0].reshape(D // 32, 16, 2).transpose(0, 2, 1).reshape(D)
    out3 = _sc_edge_head(p12, q, src2, dst2, u_de)
    return out3.reshape(2, E, 1) + us_b


# speedup vs baseline: 1.3434x; 1.0071x over previous
import functools

import jax
import jax.numpy as jnp
from jax import lax
from jax.experimental import pallas as pl
from jax.experimental.pallas import tpu as pltpu
from jax.experimental.pallas import tpu_sc as plsc

_NC = 2
_NS = 16
_LANES = 16
_CH = 80


def _sc_segsum(feat, emb, src2, dst2):
    N, D = feat.shape
    Np = -(-N // (_NS * _CH)) * (_NS * _CH)
    nrows, ch = src2.shape
    assert ch == _CH
    E = nrows * _CH
    epw = E // _NS
    nch = epw // _CH
    IB = 25
    nblk = nch // IB
    rows_pt = Np // _NS
    assert epw * _NS == E and nch * _CH == epw and rows_pt * _NS == Np
    assert nblk * IB == nch and rows_pt % _CH == 0

    mesh = plsc.VectorSubcoreMesh(core_axis_name="c", subcore_axis_name="s")

    @functools.partial(
        pl.kernel,
        out_type=[
            jax.ShapeDtypeStruct((N, D), jnp.float32),
            jax.ShapeDtypeStruct((N, D), jnp.float32),
            jax.ShapeDtypeStruct((N,), jnp.float32),
        ],
        mesh=mesh,
        scratch_types=[
            pltpu.VMEM_SHARED((Np, D), jnp.float32),
            pltpu.VMEM_SHARED((Np,), jnp.float32),
            pltpu.VMEM((IB, _CH), jnp.int32),
            pltpu.VMEM((IB, _CH), jnp.int32),
            pltpu.VMEM((3, _CH, D), jnp.float32),
            pltpu.VMEM((_CH,), jnp.float32),
            pltpu.VMEM((N,), jnp.float32),
            pltpu.SemaphoreType.DMA,
            pltpu.SemaphoreType.DMA,
            pltpu.SemaphoreType.DMA,
            pltpu.SemaphoreType.DMA,
            pltpu.SemaphoreType.DMA,
            pltpu.SemaphoreType.DMA,
            pltpu.SemaphoreType.DMA,
            pltpu.SemaphoreType.DMA,
            pltpu.SemaphoreType.DMA,
        ],
        compiler_params=pltpu.CompilerParams(use_tc_tiling_on_sc=False),
    )
    def k(feat_h, emb_h, src_h, dst_h, sfeat_h, semb_h, deg_h,
          acc_s, deg_s, srcv, dstv, rowsv, onesv, degv,
          gs0, gs1, gs2, ss0, ss1, ss2, ds0, ds1, ds2):
        cid = lax.axis_index("c")
        sid = lax.axis_index("s")
        zero16 = jnp.zeros((_LANES,), jnp.float32)
        one16 = jnp.ones((_LANES,), jnp.float32)
        gsem = (gs0, gs1, gs2)
        ssem = (ss0, ss1, ss2)
        dsem = (ds0, ds1, ds2)

        def zrow(i, _):
            for j in range(D // _LANES):
                rowsv[0, i, pl.ds(j * _LANES, _LANES)] = zero16
            return 0
        lax.fori_loop(0, _CH, zrow, 0)

        def zcp(t, _):
            off = pl.multiple_of(sid * rows_pt + t * _CH, 8)
            pltpu.sync_copy(rowsv.at[0], acc_s.at[pl.ds(off, _CH)])
            return 0
        lax.fori_loop(0, rows_pt // _CH, zcp, 0)

        for j in range(_CH // _LANES):
            onesv[pl.ds(j * _LANES, _LANES)] = one16

        @pl.when(sid == 0)
        def _():
            def zdeg(i, _):
                degv[pl.ds(i * _LANES, _LANES)] = zero16
                return 0
            lax.fori_loop(0, N // _LANES, zdeg, 0)
            pltpu.sync_copy(degv, deg_s.at[pl.ds(0, N)])

        plsc.subcore_barrier()

        def sweep(table_h, with_deg):
            def block(bi, _):
                irow = sid * nch + bi * IB
                pltpu.sync_copy(src_h.at[pl.ds(irow, IB)], srcv)
                pltpu.sync_copy(dst_h.at[pl.ds(irow, IB)], dstv)
                gath = {}
                scat = {}
                dadd = {}

                def fire_gather(j):
                    b = j % 3
                    gath[j] = pltpu.async_copy(
                        table_h.at[srcv.at[j]], rowsv.at[b], gsem[b])

                def fire_scatter(j):
                    b = j % 3
                    scat[j] = pltpu.async_copy(
                        rowsv.at[b], acc_s.at[dstv.at[j]], ssem[b], add=True)
                    if with_deg:
                        dadd[j] = pltpu.async_copy(
                            onesv, deg_s.at[dstv.at[j]], dsem[b], add=True)

                fire_gather(0)
                for j in range(IB):
                    if j >= 2:
                        scat[j - 2].wait()
                        if with_deg:
                            dadd[j - 2].wait()
                    if j + 1 < IB:
                        fire_gather(j + 1)
                    gath[j].wait()
                    fire_scatter(j)
                for j in (IB - 2, IB - 1):
                    scat[j].wait()
                    if with_deg:
                        dadd[j].wait()
                return 0
            lax.fori_loop(0, nblk, block, 0)

        @pl.when(cid == 0)
        def _():
            sweep(feat_h, True)

        @pl.when(cid == 1)
        def _():
            sweep(emb_h, False)

        plsc.subcore_barrier()

        def ocp(t, _):
            off = pl.multiple_of(sid * rows_pt + t * _CH, 8)

            @pl.when(off < N)
            def _():
                pltpu.sync_copy(acc_s.at[pl.ds(off, _CH)], rowsv.at[0])

                @pl.when(cid == 0)
                def _():
                    pltpu.sync_copy(rowsv.at[0], sfeat_h.at[pl.ds(off, _CH)])

                @pl.when(cid == 1)
                def _():
                    pltpu.sync_copy(rowsv.at[0], semb_h.at[pl.ds(off, _CH)])

            return 0
        lax.fori_loop(0, rows_pt // _CH, ocp, 0)

        @pl.when(jnp.logical_and(cid == 0, sid == 0))
        def _():
            pltpu.sync_copy(deg_s.at[pl.ds(0, N)], degv)
            pltpu.sync_copy(degv, deg_h)

    return k(feat, emb, src2, dst2)


def _tc_tables(feat_p, emb_p, sfeat, semb, degb, fcWt, fcb2, w1t, w2t, w3t, linb2):
    Np, D = feat_p.shape
    BLK = 400
    assert Np % BLK == 0

    def body(f_r, e_r, sf_r, se_r, dg_r, fw_r, fb_r, w1_r, w2_r, w3_r, lb_r,
             p12_r, q_r):
        deg = dg_r[...]
        mask = deg > 0.0
        degc = jnp.maximum(deg, 1.0)
        f = f_r[...]
        e = e_r[...]
        m1_in = jnp.dot(f + sf_r[...] / degc, fw_r[...],
                        preferred_element_type=jnp.float32) + 2.0 * fb_r[...]
        m1 = jnp.where(mask, jnp.maximum(m1_in, 0.0), 0.0)
        root1 = jnp.where(mask, se_r[...] / degc, e)
        p1 = jnp.dot(e, w1_r[...], preferred_element_type=jnp.float32)
        p2 = (jnp.dot(root1, w1_r[...], preferred_element_type=jnp.float32)
              + jnp.dot(m1, w2_r[...], preferred_element_type=jnp.float32))
        p12_r[...] = jnp.concatenate([p1, p2], axis=1).astype(jnp.bfloat16)
        q_r[...] = (jnp.dot(f, w3_r[...], preferred_element_type=jnp.float32)
                    + lb_r[...]).astype(jnp.bfloat16)

    row_spec = pl.BlockSpec((BLK, D), lambda i: (i, 0))
    w_spec = pl.BlockSpec((D, D), lambda i: (0, 0))
    b_spec = pl.BlockSpec((1, D), lambda i: (0, 0))
    return pl.pallas_call(
        body,
        grid=(Np // BLK,),
        in_specs=[row_spec, row_spec, row_spec, row_spec, row_spec,
                  w_spec, b_spec, w_spec, w_spec, w_spec, b_spec],
        out_specs=[pl.BlockSpec((BLK, 2 * D), lambda i: (i, 0)), row_spec],
        out_shape=[
            jax.ShapeDtypeStruct((Np, 2 * D), jnp.bfloat16),
            jax.ShapeDtypeStruct((Np, D), jnp.bfloat16),
        ],
    )(feat_p, emb_p, sfeat, semb, degb, fcWt, fcb2, w1t, w2t, w3t, linb2)


def _sc_edge_head(p12, q, src2, dst2, u):
    Np, D2 = p12.shape
    D = q.shape[1]
    nrows, ch = src2.shape
    assert ch == _CH
    E = nrows * _CH
    NW = _NC * _NS
    nch = nrows // NW
    IBC = 5
    nblk = nch // IBC
    assert nch * NW == nrows and nblk * IBC == nch

    mesh = plsc.VectorSubcoreMesh(core_axis_name="c", subcore_axis_name="s")

    @functools.partial(
        pl.kernel,
        out_type=jax.ShapeDtypeStruct((2, nrows, _CH), jnp.float32),
        mesh=mesh,
        scratch_types=[
            pltpu.VMEM((IBC, _CH), jnp.int32),
            pltpu.VMEM((IBC, _CH), jnp.int32),
            pltpu.VMEM((3, _CH, D2), jnp.bfloat16),
            pltpu.VMEM((3, _CH, D), jnp.bfloat16),
            pltpu.VMEM((D,), jnp.float32),
            pltpu.VMEM((IBC, _CH), jnp.float32),
            pltpu.VMEM((IBC, _CH), jnp.float32),
            pltpu.SemaphoreType.DMA,
            pltpu.SemaphoreType.DMA,
            pltpu.SemaphoreType.DMA,
            pltpu.SemaphoreType.DMA,
            pltpu.SemaphoreType.DMA,
            pltpu.SemaphoreType.DMA,
        ],
        compiler_params=pltpu.CompilerParams(use_tc_tiling_on_sc=False,
                                            needs_layout_passes=False),
    )
    def k(p_h, q_h, src_h, dst_h, u_h, out_h,
          srcv, dstv, pv, qv, uv, o1v, o2v,
          ps0, ps1, ps2, qs0, qs1, qs2):
        cid = lax.axis_index("c")
        sid = lax.axis_index("s")
        wid = sid * _NC + cid
        psem = (ps0, ps1, ps2)
        qsem = (qs0, qs1, qs2)
        pltpu.sync_copy(u_h, uv)
        uregs = [uv[pl.ds(j * _LANES, _LANES)] for j in range(D // _LANES)]
        lane = lax.iota(jnp.int32, _LANES)

        def block(bi, _):
            irow = wid * nch + bi * IBC
            pltpu.sync_copy(src_h.at[pl.ds(irow, IBC)], srcv)
            pltpu.sync_copy(dst_h.at[pl.ds(irow, IBC)], dstv)
            gp = {}
            gq = {}

            def fire(j):
                b = j % 3
                gp[j] = pltpu.async_copy(p_h.at[srcv.at[j]], pv.at[b], psem[b])
                gq[j] = pltpu.async_copy(q_h.at[dstv.at[j]], qv.at[b], qsem[b])

            fire(0)
            fire(1)
            for j in range(IBC):
                b = j % 3
                if j + 2 < IBC:
                    fire(j + 2)
                gp[j].wait()
                gq[j].wait()

                def group(g, _, b=b, j=j):
                    row0 = pl.multiple_of(g * _LANES, 8)
                    res1 = jnp.zeros((_LANES,), jnp.float32)
                    res2 = jnp.zeros((_LANES,), jnp.float32)
                    W = 2 * _LANES
                    for e16 in range(_LANES):
                        r = row0 + e16
                        qr = qv.at[b, r]
                        pr = pv.at[b, r]
                        t1 = []
                        t2 = []
                        for t in range(D // W):
                            qa, qb = plsc.unpack(qr[pl.ds(t * W, W)],
                                                 format=plsc.PackFormat.INTERLEAVED)
                            pa, pb = plsc.unpack(pr[pl.ds(t * W, W)],
                                                 format=plsc.PackFormat.INTERLEAVED)
                            ra, rb = plsc.unpack(pr[pl.ds(D + t * W, W)],
                                                 format=plsc.PackFormat.INTERLEAVED)
                            ue = uregs[2 * t]
                            uo = uregs[2 * t + 1]
                            t1.append(jnp.maximum(pa + qa, 0.0) * ue)
                            t1.append(jnp.maximum(pb + qb, 0.0) * uo)
                            t2.append(jnp.maximum(ra + qa, 0.0) * ue)
                            t2.append(jnp.maximum(rb + qb, 0.0) * uo)
                        while len(t1) > 1:
                            t1 = [t1[i] + t1[i + 1] for i in range(0, len(t1), 2)]
                            t2 = [t2[i] + t2[i + 1] for i in range(0, len(t2), 2)]
                        res1 = jnp.where(lane == e16, jnp.sum(t1[0]), res1)
                        res2 = jnp.where(lane == e16, jnp.sum(t2[0]), res2)
                    o1v[j, pl.ds(row0, _LANES)] = res1
                    o2v[j, pl.ds(row0, _LANES)] = res2
                    return 0
                lax.fori_loop(0, _CH // _LANES, group, 0)

            pltpu.sync_copy(o1v, out_h.at[0, pl.ds(irow, IBC)])
            pltpu.sync_copy(o2v, out_h.at[1, pl.ds(irow, IBC)])
            return 0
        lax.fori_loop(0, nblk, block, 0)

    return k(p12, q, src2, dst2, u)


def kernel(features, emb, fc_W, fc_b, lin_W, lin_b, us_W, us_b, edge_index):
    N, D = features.shape
    H = emb.shape[1]
    E = edge_index.shape[1]

    src2 = edge_index[0].reshape(E // _CH, _CH)
    dst2 = edge_index[1].reshape(E // _CH, _CH)

    sfeat, semb, deg = _sc_segsum(features, emb, src2, dst2)
    degb = jnp.broadcast_to(deg[:, None], (N, D))

    w1t = lin_W[:, :H].T
    w2t = lin_W[:, H:2 * H].T
    w3t = lin_W[:, 2 * H:].T
    p12, q = _tc_tables(features, emb, sfeat, semb, degb,
                        fc_W.T, fc_b[None, :], w1t, w2t, w3t, lin_b[None, :])

    u_de = us_W[0].reshape(D // 32, 16, 2).transpose(0, 2, 1).reshape(D)
    out3 = _sc_edge_head(p12, q, src2, dst2, u_de)
    return out3.reshape(2, E, 1) + us_b
